# baseline (speedup 1.0000x reference)
import functools

import jax
import jax.numpy as jnp
import numpy as np
from jax import lax
from jax.experimental import pallas as pl
from jax.experimental.pallas import tpu as pltpu
from jax.experimental.pallas import tpu_sc as plsc

N_NODES = 10000
N_EDGES = 160000
IN_DIM = 1152
HID_DIM = 768
EDGE_DIM = 16
EDGE_HID = 64

_M_BLK = 1000


def _pack_halves(h):
    half = h.shape[-1] // 2
    a = jax.lax.bitcast_convert_type(
        h[:, :half].astype(jnp.bfloat16), jnp.uint16).astype(jnp.uint32)
    b = jax.lax.bitcast_convert_type(
        h[:, half:].astype(jnp.bfloat16), jnp.uint16).astype(jnp.uint32)
    return jax.lax.bitcast_convert_type(a | (b << 16), jnp.int32)


def _node_body(nf, fii, wa, wb, w1, w2, fii_out, h1_out, h2_out):
    x = nf[...]
    a = jnp.dot(x, wa[...], preferred_element_type=jnp.float32)
    b = jnp.dot(x, wb[...], preferred_element_type=jnp.float32)
    fii_out[...] = fii[...] + a * jax.nn.sigmoid(a) * b
    h1_out[...] = _pack_halves(
        jnp.dot(x, w1[...], preferred_element_type=jnp.float32))
    h2_out[...] = _pack_halves(
        jnp.dot(x, w2[...], preferred_element_type=jnp.float32))


def _node_kernel(node_feat, fii, wa, wb, w1, w2):
    grid = (N_NODES // _M_BLK,)
    out_shape = [
        jax.ShapeDtypeStruct((N_NODES, HID_DIM), jnp.float32),
        jax.ShapeDtypeStruct((N_NODES, HID_DIM // 2), jnp.int32),
        jax.ShapeDtypeStruct((N_NODES, HID_DIM // 2), jnp.int32),
    ]
    return pl.pallas_call(
        _node_body,
        grid=grid,
        in_specs=[
            pl.BlockSpec((_M_BLK, IN_DIM), lambda i: (i, 0)),
            pl.BlockSpec((_M_BLK, HID_DIM), lambda i: (i, 0)),
            pl.BlockSpec((IN_DIM, HID_DIM), lambda i: (0, 0)),
            pl.BlockSpec((IN_DIM, HID_DIM), lambda i: (0, 0)),
            pl.BlockSpec((IN_DIM, HID_DIM), lambda i: (0, 0)),
            pl.BlockSpec((IN_DIM, HID_DIM), lambda i: (0, 0)),
        ],
        out_specs=[
            pl.BlockSpec((_M_BLK, HID_DIM), lambda i: (i, 0)),
            pl.BlockSpec((_M_BLK, HID_DIM // 2), lambda i: (i, 0)),
            pl.BlockSpec((_M_BLK, HID_DIM // 2), lambda i: (i, 0)),
        ],
        out_shape=out_shape,
    )(node_feat, fii, wa, wb, w1, w2)


_E_BLK = 2000


def _gate_body(ea, we1, be1, we2, be2, gate_out):
    t = jnp.dot(ea[...], we1[...], preferred_element_type=jnp.float32) + be1[...]
    t = t * jax.nn.sigmoid(t)
    gate_out[...] = _pack_halves(
        jnp.dot(t, we2[...], preferred_element_type=jnp.float32) + be2[...])


def _gate_kernel(edge_attr, we1, be1, we2, be2):
    grid = (N_EDGES // _E_BLK,)
    return pl.pallas_call(
        _gate_body,
        grid=grid,
        in_specs=[
            pl.BlockSpec((_E_BLK, EDGE_DIM), lambda i: (i, 0)),
            pl.BlockSpec((EDGE_DIM, EDGE_HID), lambda i: (0, 0)),
            pl.BlockSpec((1, EDGE_HID), lambda i: (0, 0)),
            pl.BlockSpec((EDGE_HID, HID_DIM), lambda i: (0, 0)),
            pl.BlockSpec((1, HID_DIM), lambda i: (0, 0)),
        ],
        out_specs=pl.BlockSpec((_E_BLK, HID_DIM // 2), lambda i: (i, 0)),
        out_shape=jax.ShapeDtypeStruct((N_EDGES, HID_DIM // 2), jnp.int32),
    )(edge_attr, we1, be1, we2, be2)


_NC = 2
_NS = 16
_NW = _NC * _NS
_E_PER_W = N_EDGES // _NW
_B = 8
_NBUF = 4
_N_CHUNKS = _E_PER_W // _B
_N_GROUPS = (_N_CHUNKS + _NBUF - 1) // _NBUF
_LANES = 16
_COLS = HID_DIM // _LANES
_PAIRS = HID_DIM // (2 * _LANES)

_HALF = HID_DIM // 2

_K = np.arange(HID_DIM)
_PERM = np.where(
    _K < _HALF,
    32 * (_K // 16) + _K % 16,
    32 * ((_K - _HALF) // 16) + 16 + (_K - _HALF) % 16,
)


def _edge_body(h1_hbm, h2_hbm, gate_hbm, fij_hbm, src_hbm, dst_hbm, out_hbm,
               src_all, dst_all, r1, r2, gv, fv,
               si0, si1, si2, si3, so0, so1, so2, so3):
    sem_in = [si0, si1, si2, si3]
    sem_out = [so0, so1, so2, so3]
    wid = lax.axis_index("s") * _NC + lax.axis_index("c")
    base0 = wid * _E_PER_W
    pltpu.sync_copy(src_hbm.at[pl.ds(base0, _E_PER_W)], src_all)
    pltpu.sync_copy(dst_hbm.at[pl.ds(base0, _E_PER_W)], dst_all)

    def in_copies(c, b):
        base = base0 + c * _B
        return [
            pltpu.make_async_copy(
                h1_hbm.at[src_all.at[pl.ds(c * _B, _B)]], r1.at[b], sem_in[b]),
            pltpu.make_async_copy(
                h2_hbm.at[dst_all.at[pl.ds(c * _B, _B)]], r2.at[b], sem_in[b]),
            pltpu.make_async_copy(
                gate_hbm.at[pl.ds(base, _B), :], gv.at[b], sem_in[b]),
            pltpu.make_async_copy(
                fij_hbm.at[pl.ds(base, _B), :], fv.at[b], sem_in[b]),
        ]

    def out_copy(c, b):
        base = base0 + c * _B
        return pltpu.make_async_copy(
            fv.at[b], out_hbm.at[pl.ds(base, _B), :], sem_out[b])

    for b in range(_NBUF - 1):
        for cp in in_copies(b, b):
            cp.start()

    def group_body(g, carry):
        for b in range(_NBUF):
            c = g * _NBUF + b

            @pl.when(c < _N_CHUNKS)
            def _():
                for cp in in_copies(c, b):
                    cp.wait()

                def pair_body(j, carry2):
                    jw = pl.multiple_of(j * _LANES, _LANES)
                    jo = pl.multiple_of(j * 2 * _LANES, 2 * _LANES)
                    jo2 = pl.multiple_of(j * 2 * _LANES + _LANES, _LANES)
                    mask = jnp.int32(-65536)
                    for i in range(_B):
                        sl = pl.ds(jw, _LANES)
                        u1 = r1[b, i, sl]
                        u2 = r2[b, i, sl]
                        ug = gv[b, i, sl]
                        bc = jax.lax.bitcast_convert_type
                        lo = (bc(u1 << 16, jnp.float32)
                              * bc(u2 << 16, jnp.float32)
                              * bc(ug << 16, jnp.float32))
                        hi = (bc(u1 & mask, jnp.float32)
                              * bc(u2 & mask, jnp.float32)
                              * bc(ug & mask, jnp.float32))
                        plsc.addupdate(fv.at[b, i, pl.ds(jo, _LANES)], lo)
                        plsc.addupdate(fv.at[b, i, pl.ds(jo2, _LANES)], hi)
                    return carry2

                lax.fori_loop(0, _PAIRS, pair_body, 0)
                out_copy(c, b).start()

            c2 = c + _NBUF - 1
            b2 = (b + _NBUF - 1) % _NBUF

            @pl.when((c2 >= _NBUF) & (c2 < _N_CHUNKS))
            def _():
                out_copy(c2 - _NBUF, b2).wait()

            @pl.when(c2 < _N_CHUNKS)
            def _():
                for cp in in_copies(c2, b2):
                    cp.start()

        return carry

    lax.fori_loop(0, _N_GROUPS, group_body, 0)
    for b in range(_NBUF):
        last_c = _N_CHUNKS - 1 - ((_N_CHUNKS - 1 - b) % _NBUF)
        out_copy(last_c, b).wait()


def _edge_kernel(h1, h2, gate, fij, src, dst):
    mesh = plsc.VectorSubcoreMesh(core_axis_name="c", subcore_axis_name="s")
    f = functools.partial(
        pl.kernel,
        out_type=jax.ShapeDtypeStruct((N_EDGES, HID_DIM), jnp.float32),
        mesh=mesh,
        scratch_types=[
            pltpu.VMEM((_E_PER_W,), jnp.int32),
            pltpu.VMEM((_E_PER_W,), jnp.int32),
            pltpu.VMEM((_NBUF, _B, _HALF), jnp.int32),
            pltpu.VMEM((_NBUF, _B, _HALF), jnp.int32),
            pltpu.VMEM((_NBUF, _B, _HALF), jnp.int32),
            pltpu.VMEM((_NBUF, _B, HID_DIM), jnp.float32),
        ] + [pltpu.SemaphoreType.DMA] * (2 * _NBUF),
    )(_edge_body)
    return f(h1, h2, gate, fij, src, dst)


def kernel(node_feat, edge_attr, edge_index, fii, fij,
           W_self_a, W_self_b, W1, W2, We1, be1, We2, be2):
    src = edge_index[0].astype(jnp.int32)
    dst = edge_index[1].astype(jnp.int32)
    perm = jnp.asarray(_PERM, dtype=jnp.int32)
    w1p = jnp.take(W1, perm, axis=1)
    w2p = jnp.take(W2, perm, axis=1)
    we2p = jnp.take(We2, perm, axis=1)
    be2p = jnp.take(be2, perm, axis=0)
    fii_out, h1, h2 = _node_kernel(node_feat, fii, W_self_a, W_self_b, w1p, w2p)
    gate = _gate_kernel(
        edge_attr, We1, be1.reshape(1, EDGE_HID), we2p, be2p.reshape(1, HID_DIM)
    )
    fij_out = _edge_kernel(h1, h2, gate, fij, src, dst)
    return (fii_out, fij_out)


# speedup vs baseline: 1.0241x; 1.0241x over previous
import functools

import jax
import jax.numpy as jnp
import numpy as np
from jax import lax
from jax.experimental import pallas as pl
from jax.experimental.pallas import tpu as pltpu
from jax.experimental.pallas import tpu_sc as plsc

N_NODES = 10000
N_EDGES = 160000
IN_DIM = 1152
HID_DIM = 768
EDGE_DIM = 16
EDGE_HID = 64

_M_BLK = 1000


def _pack_halves(h):
    half = h.shape[-1] // 2
    a = jax.lax.bitcast_convert_type(
        h[:, :half].astype(jnp.bfloat16), jnp.uint16).astype(jnp.uint32)
    b = jax.lax.bitcast_convert_type(
        h[:, half:].astype(jnp.bfloat16), jnp.uint16).astype(jnp.uint32)
    return jax.lax.bitcast_convert_type(a | (b << 16), jnp.int32)


def _h_body(nf, w1, w2, h1_out, h2_out):
    x = nf[...]
    h1_out[...] = _pack_halves(
        jnp.dot(x, w1[...], preferred_element_type=jnp.float32))
    h2_out[...] = _pack_halves(
        jnp.dot(x, w2[...], preferred_element_type=jnp.float32))


def _h_kernel(node_feat, w1, w2):
    grid = (N_NODES // _M_BLK,)
    out_shape = [
        jax.ShapeDtypeStruct((N_NODES, HID_DIM // 2), jnp.int32),
        jax.ShapeDtypeStruct((N_NODES, HID_DIM // 2), jnp.int32),
    ]
    return pl.pallas_call(
        _h_body,
        grid=grid,
        in_specs=[
            pl.BlockSpec((_M_BLK, IN_DIM), lambda i: (i, 0)),
            pl.BlockSpec((IN_DIM, HID_DIM), lambda i: (0, 0)),
            pl.BlockSpec((IN_DIM, HID_DIM), lambda i: (0, 0)),
        ],
        out_specs=[
            pl.BlockSpec((_M_BLK, HID_DIM // 2), lambda i: (i, 0)),
            pl.BlockSpec((_M_BLK, HID_DIM // 2), lambda i: (i, 0)),
        ],
        out_shape=out_shape,
    )(node_feat, w1, w2)


def _fii_body(nf, fii, wa, wb, fii_out):
    x = nf[...]
    a = jnp.dot(x, wa[...], preferred_element_type=jnp.float32)
    b = jnp.dot(x, wb[...], preferred_element_type=jnp.float32)
    fii_out[...] = fii[...] + a * jax.nn.sigmoid(a) * b


def _fii_kernel(node_feat, fii, wa, wb):
    grid = (N_NODES // _M_BLK,)
    return pl.pallas_call(
        _fii_body,
        grid=grid,
        in_specs=[
            pl.BlockSpec((_M_BLK, IN_DIM), lambda i: (i, 0)),
            pl.BlockSpec((_M_BLK, HID_DIM), lambda i: (i, 0)),
            pl.BlockSpec((IN_DIM, HID_DIM), lambda i: (0, 0)),
            pl.BlockSpec((IN_DIM, HID_DIM), lambda i: (0, 0)),
        ],
        out_specs=pl.BlockSpec((_M_BLK, HID_DIM), lambda i: (i, 0)),
        out_shape=jax.ShapeDtypeStruct((N_NODES, HID_DIM), jnp.float32),
    )(node_feat, fii, wa, wb)


_E_BLK = 2000


def _gate_body(ea, we1, be1, we2, be2, gate_out):
    t = jnp.dot(ea[...], we1[...], preferred_element_type=jnp.float32) + be1[...]
    t = t * jax.nn.sigmoid(t)
    gate_out[...] = _pack_halves(
        jnp.dot(t, we2[...], preferred_element_type=jnp.float32) + be2[...])


def _gate_kernel(edge_attr, we1, be1, we2, be2):
    grid = (N_EDGES // _E_BLK,)
    return pl.pallas_call(
        _gate_body,
        grid=grid,
        in_specs=[
            pl.BlockSpec((_E_BLK, EDGE_DIM), lambda i: (i, 0)),
            pl.BlockSpec((EDGE_DIM, EDGE_HID), lambda i: (0, 0)),
            pl.BlockSpec((1, EDGE_HID), lambda i: (0, 0)),
            pl.BlockSpec((EDGE_HID, HID_DIM), lambda i: (0, 0)),
            pl.BlockSpec((1, HID_DIM), lambda i: (0, 0)),
        ],
        out_specs=pl.BlockSpec((_E_BLK, HID_DIM // 2), lambda i: (i, 0)),
        out_shape=jax.ShapeDtypeStruct((N_EDGES, HID_DIM // 2), jnp.int32),
    )(edge_attr, we1, be1, we2, be2)


_NC = 2
_NS = 16
_NW = _NC * _NS
_E_PER_W = N_EDGES // _NW
_B = 8
_NBUF = 4
_N_CHUNKS = _E_PER_W // _B
_N_GROUPS = (_N_CHUNKS + _NBUF - 1) // _NBUF
_LANES = 16
_COLS = HID_DIM // _LANES
_PAIRS = HID_DIM // (2 * _LANES)

_HALF = HID_DIM // 2

_K = np.arange(HID_DIM)
_PERM = np.where(
    _K < _HALF,
    32 * (_K // 16) + _K % 16,
    32 * ((_K - _HALF) // 16) + 16 + (_K - _HALF) % 16,
)


def _edge_body(h1_hbm, h2_hbm, gate_hbm, fij_hbm, src_hbm, dst_hbm, out_hbm,
               src_all, dst_all, r1, r2, gv, fv,
               si0, si1, si2, si3, so0, so1, so2, so3):
    sem_in = [si0, si1, si2, si3]
    sem_out = [so0, so1, so2, so3]
    wid = lax.axis_index("s") * _NC + lax.axis_index("c")
    base0 = wid * _E_PER_W
    pltpu.sync_copy(src_hbm.at[pl.ds(base0, _E_PER_W)], src_all)
    pltpu.sync_copy(dst_hbm.at[pl.ds(base0, _E_PER_W)], dst_all)

    def in_copies(c, b):
        base = base0 + c * _B
        return [
            pltpu.make_async_copy(
                h1_hbm.at[src_all.at[pl.ds(c * _B, _B)]], r1.at[b], sem_in[b]),
            pltpu.make_async_copy(
                h2_hbm.at[dst_all.at[pl.ds(c * _B, _B)]], r2.at[b], sem_in[b]),
            pltpu.make_async_copy(
                gate_hbm.at[pl.ds(base, _B), :], gv.at[b], sem_in[b]),
            pltpu.make_async_copy(
                fij_hbm.at[pl.ds(base, _B), :], fv.at[b], sem_in[b]),
        ]

    def out_copy(c, b):
        base = base0 + c * _B
        return pltpu.make_async_copy(
            fv.at[b], out_hbm.at[pl.ds(base, _B), :], sem_out[b])

    for b in range(_NBUF - 1):
        for cp in in_copies(b, b):
            cp.start()

    def group_body(g, carry):
        for b in range(_NBUF):
            c = g * _NBUF + b

            @pl.when(c < _N_CHUNKS)
            def _():
                for cp in in_copies(c, b):
                    cp.wait()

                def pair_body(j, carry2):
                    jw = pl.multiple_of(j * _LANES, _LANES)
                    jo = pl.multiple_of(j * 2 * _LANES, 2 * _LANES)
                    jo2 = pl.multiple_of(j * 2 * _LANES + _LANES, _LANES)
                    mask = jnp.int32(-65536)
                    for i in range(_B):
                        sl = pl.ds(jw, _LANES)
                        u1 = r1[b, i, sl]
                        u2 = r2[b, i, sl]
                        ug = gv[b, i, sl]
                        bc = jax.lax.bitcast_convert_type
                        lo = (bc(u1 << 16, jnp.float32)
                              * bc(u2 << 16, jnp.float32)
                              * bc(ug << 16, jnp.float32))
                        hi = (bc(u1 & mask, jnp.float32)
                              * bc(u2 & mask, jnp.float32)
                              * bc(ug & mask, jnp.float32))
                        plsc.addupdate(fv.at[b, i, pl.ds(jo, _LANES)], lo)
                        plsc.addupdate(fv.at[b, i, pl.ds(jo2, _LANES)], hi)
                    return carry2

                lax.fori_loop(0, _PAIRS, pair_body, 0)
                out_copy(c, b).start()

            c2 = c + _NBUF - 1
            b2 = (b + _NBUF - 1) % _NBUF

            @pl.when((c2 >= _NBUF) & (c2 < _N_CHUNKS))
            def _():
                out_copy(c2 - _NBUF, b2).wait()

            @pl.when(c2 < _N_CHUNKS)
            def _():
                for cp in in_copies(c2, b2):
                    cp.start()

        return carry

    lax.fori_loop(0, _N_GROUPS, group_body, 0)
    for b in range(_NBUF):
        last_c = _N_CHUNKS - 1 - ((_N_CHUNKS - 1 - b) % _NBUF)
        out_copy(last_c, b).wait()


def _edge_kernel(h1, h2, gate, fij, src, dst):
    mesh = plsc.VectorSubcoreMesh(core_axis_name="c", subcore_axis_name="s")
    f = functools.partial(
        pl.kernel,
        out_type=jax.ShapeDtypeStruct((N_EDGES, HID_DIM), jnp.float32),
        mesh=mesh,
        scratch_types=[
            pltpu.VMEM((_E_PER_W,), jnp.int32),
            pltpu.VMEM((_E_PER_W,), jnp.int32),
            pltpu.VMEM((_NBUF, _B, _HALF), jnp.int32),
            pltpu.VMEM((_NBUF, _B, _HALF), jnp.int32),
            pltpu.VMEM((_NBUF, _B, _HALF), jnp.int32),
            pltpu.VMEM((_NBUF, _B, HID_DIM), jnp.float32),
        ] + [pltpu.SemaphoreType.DMA] * (2 * _NBUF),
    )(_edge_body)
    return f(h1, h2, gate, fij, src, dst)


def kernel(node_feat, edge_attr, edge_index, fii, fij,
           W_self_a, W_self_b, W1, W2, We1, be1, We2, be2):
    src = edge_index[0].astype(jnp.int32)
    dst = edge_index[1].astype(jnp.int32)
    perm = jnp.asarray(_PERM, dtype=jnp.int32)
    w1p = jnp.take(W1, perm, axis=1)
    w2p = jnp.take(W2, perm, axis=1)
    we2p = jnp.take(We2, perm, axis=1)
    be2p = jnp.take(be2, perm, axis=0)
    h1, h2 = _h_kernel(node_feat, w1p, w2p)
    gate = _gate_kernel(
        edge_attr, We1, be1.reshape(1, EDGE_HID), we2p, be2p.reshape(1, HID_DIM)
    )
    fij_out = _edge_kernel(h1, h2, gate, fij, src, dst)
    fii_out = _fii_kernel(node_feat, fii, W_self_a, W_self_b)
    return (fii_out, fij_out)
